# baseline (speedup 1.0000x reference)
import functools

import jax
import jax.numpy as jnp
from jax import lax
from jax.experimental import pallas as pl
from jax.experimental.pallas import tpu as pltpu
from jax.experimental.pallas import tpu_sc as plsc

B, S, H, E, TOP_K = 1, 2048, 1024, 8, 2
F = 4 * H

BM = 256
NB = (S * TOP_K) // BM + E
N_SLOTS = NB * BM
RCH = 512
NCH = (S * TOP_K) // RCH

NP = S * TOP_K
NS = 16
CHUNK = NP // NS
HC = CHUNK // 2
SLOT_CH = N_SLOTS // NS


def _scatter_sc(pos_hbm, wf_hbm,
                rows_hbm, wv_hbm,
                pos_a, pos_b, wv_a, wv_b, rv_a, rv_b,
                zb_ref, zbf_ref, sh_rows, sh_wv):
    wid = lax.axis_index("s")
    lanes = lax.iota(jnp.int32, 16)

    pltpu.sync_copy(pos_hbm.at[pl.ds(wid * CHUNK, HC)], pos_a)
    pltpu.sync_copy(pos_hbm.at[pl.ds(wid * CHUNK + HC, HC)], pos_b)
    pltpu.sync_copy(wf_hbm.at[pl.ds(wid * CHUNK, HC)], wv_a)
    pltpu.sync_copy(wf_hbm.at[pl.ds(wid * CHUNK + HC, HC)], wv_b)

    for v in range(HC // 16):
        j0 = wid * CHUNK + v * 16
        rv_a[pl.ds(v * 16, 16)] = (j0 + lanes) >> 1
        rv_b[pl.ds(v * 16, 16)] = (j0 + HC + lanes) >> 1

    for v in range(SLOT_CH // 16):
        zb_ref[pl.ds(v * 16, 16)] = jnp.zeros((16,), jnp.int32)
        zbf_ref[pl.ds(v * 16, 16)] = jnp.zeros((16,), jnp.float32)
    pltpu.sync_copy(zb_ref, sh_rows.at[pl.ds(wid * SLOT_CH, SLOT_CH)])
    pltpu.sync_copy(zbf_ref, sh_wv.at[pl.ds(wid * SLOT_CH, SLOT_CH)])
    plsc.subcore_barrier()

    pltpu.sync_copy(rv_a, sh_rows.at[pos_a], add=True)
    pltpu.sync_copy(rv_b, sh_rows.at[pos_b], add=True)
    pltpu.sync_copy(wv_a, sh_wv.at[pos_a], add=True)
    pltpu.sync_copy(wv_b, sh_wv.at[pos_b], add=True)
    plsc.subcore_barrier()

    pltpu.sync_copy(sh_rows.at[pl.ds(wid * SLOT_CH, SLOT_CH)],
                    rows_hbm.at[pl.ds(wid * SLOT_CH, SLOT_CH)])
    pltpu.sync_copy(sh_wv.at[pl.ds(wid * SLOT_CH, SLOT_CH)],
                    wv_hbm.at[pl.ds(wid * SLOT_CH, SLOT_CH)])


_scatter = functools.partial(
    pl.kernel,
    out_type=(
        jax.ShapeDtypeStruct((N_SLOTS,), jnp.int32),
        jax.ShapeDtypeStruct((N_SLOTS,), jnp.float32),
    ),
    mesh=plsc.VectorSubcoreMesh(
        core_axis_name="c", subcore_axis_name="s", num_cores=1),
    scratch_types=(
        pltpu.VMEM((HC,), jnp.int32),
        pltpu.VMEM((HC,), jnp.int32),
        pltpu.VMEM((HC,), jnp.float32),
        pltpu.VMEM((HC,), jnp.float32),
        pltpu.VMEM((HC,), jnp.int32),
        pltpu.VMEM((HC,), jnp.int32),
        pltpu.VMEM((SLOT_CH,), jnp.int32),
        pltpu.VMEM((SLOT_CH,), jnp.float32),
        pltpu.VMEM_SHARED((N_SLOTS,), jnp.int32),
        pltpu.VMEM_SHARED((N_SLOTS,), jnp.float32),
    ),
)(_scatter_sc)


def _moe_block_kernel(eids_ref, rows_ref, nact_ref,
                      x_ref, w1_ref, b1_ref, w2_ref, b2_ref, wmat_ref,
                      out_ref, xg_ref, yacc_ref):
    b = pl.program_id(0)

    @pl.when(b == 0)
    def _init_out():
        out_ref[...] = jnp.zeros_like(out_ref)

    @pl.when(b < nact_ref[0])
    def _active():
        def gather(i, _):
            r = rows_ref[b * BM + i]
            xg_ref[pl.ds(i, 1), :] = x_ref[pl.ds(r, 1), :]
            return 0
        lax.fori_loop(0, BM, gather, 0, unroll=8)

        h = lax.dot_general(
            xg_ref[...].astype(jnp.bfloat16), w1_ref[0],
            (((1,), (1,)), ((), ())), preferred_element_type=jnp.float32)
        h = h + b1_ref[0, 0]
        h = 0.5 * h * (1.0 + lax.erf(h * 0.7071067811865476))
        y = lax.dot_general(
            h.astype(jnp.bfloat16), w2_ref[0],
            (((1,), (1,)), ((), ())), preferred_element_type=jnp.float32)
        yacc_ref[...] = (y + b2_ref[0]) * wmat_ref[0]

        def scatter(i, _):
            r = rows_ref[b * BM + i]
            out_ref[pl.ds(r, 1), :] += yacc_ref[pl.ds(i, 1), :]
            return 0
        lax.fori_loop(0, BM, scatter, 0, unroll=8)


@functools.partial(jax.jit, static_argnums=())
def kernel(x, Wg, W1, b1, W2, b2):
    logits = jnp.einsum('bsh,eh->bse', jax.lax.stop_gradient(x), Wg)
    probs = jax.nn.softmax(logits, axis=-1)
    top_k_weights, top_k_indices = jax.lax.top_k(probs, TOP_K)

    e_flat = top_k_indices.reshape(NP).astype(jnp.int32)
    w_flat = top_k_weights.reshape(NP)
    onehot = (e_flat[:, None] == jnp.arange(E, dtype=jnp.int32)[None, :]
              ).astype(jnp.float32)
    ohc = onehot.reshape(NCH, RCH, E)
    ii = jax.lax.broadcasted_iota(jnp.int32, (RCH, RCH), 0)
    jj = jax.lax.broadcasted_iota(jnp.int32, (RCH, RCH), 1)
    ltri = (ii > jj).astype(jnp.float32)
    local_rank = jnp.einsum('ij,cje->cie', ltri, ohc,
                            preferred_element_type=jnp.float32)
    chunk_tot = ohc.sum(axis=1)
    chunk_pfx = jnp.concatenate(
        [jnp.zeros((1, E), jnp.float32), jnp.cumsum(chunk_tot[:-1], axis=0)])
    rank = (local_rank + chunk_pfx[:, None, :]).reshape(NP, E)
    counts = chunk_tot.sum(axis=0)
    bpe = jnp.ceil(counts / BM)
    block_start = jnp.concatenate(
        [jnp.zeros((1,), jnp.float32), jnp.cumsum(bpe)[:-1]])
    nactive = (block_start[E - 1] + bpe[E - 1]).astype(jnp.int32)
    slot_start = block_start * BM
    pos = ((slot_start[None, :] + rank) * onehot).sum(axis=1).astype(jnp.int32)
    eids = (jnp.arange(NB, dtype=jnp.float32)[:, None]
            >= block_start[None, 1:]).sum(axis=1).astype(jnp.int32)

    rows, wvals = _scatter(pos, w_flat)
    wmat = wvals.reshape(NB, BM, 1)

    x2 = x.reshape(S, H)
    W1b = W1.astype(jnp.bfloat16)
    W2b = W2.astype(jnp.bfloat16)
    b1r = b1.reshape(E, 1, F)
    b2r = b2.reshape(E, 1, H)

    grid_spec = pltpu.PrefetchScalarGridSpec(
        num_scalar_prefetch=3,
        grid=(NB,),
        in_specs=[
            pl.BlockSpec((S, H), lambda b, eids, rows, nact: (0, 0)),
            pl.BlockSpec((1, F, H), lambda b, eids, rows, nact: (eids[b], 0, 0)),
            pl.BlockSpec((1, 1, F), lambda b, eids, rows, nact: (eids[b], 0, 0)),
            pl.BlockSpec((1, H, F), lambda b, eids, rows, nact: (eids[b], 0, 0)),
            pl.BlockSpec((1, 1, H), lambda b, eids, rows, nact: (eids[b], 0, 0)),
            pl.BlockSpec((1, BM, 1), lambda b, eids, rows, nact: (b, 0, 0)),
        ],
        out_specs=pl.BlockSpec((S, H), lambda b, eids, rows, nact: (0, 0)),
        scratch_shapes=[
            pltpu.VMEM((BM, H), jnp.float32),
            pltpu.VMEM((BM, H), jnp.float32),
        ],
    )

    out = pl.pallas_call(
        _moe_block_kernel,
        grid_spec=grid_spec,
        out_shape=jax.ShapeDtypeStruct((S, H), jnp.float32),
        compiler_params=pltpu.CompilerParams(
            dimension_semantics=("arbitrary",),
            vmem_limit_bytes=63 * 1024 * 1024,
        ),
    )(eids, rows, nactive.reshape(1), x2, W1b, b1r, W2b, b2r, wmat)

    return out.reshape(B, S, H)


# speedup vs baseline: 1.0081x; 1.0081x over previous
import functools

import jax
import jax.numpy as jnp
from jax import lax
from jax.experimental import pallas as pl
from jax.experimental.pallas import tpu as pltpu
from jax.experimental.pallas import tpu_sc as plsc

B, S, H, E, TOP_K = 1, 2048, 1024, 8, 2
F = 4 * H

BM = 256
NB = (S * TOP_K) // BM + E
N_SLOTS = NB * BM
RCH = 512
NCH = (S * TOP_K) // RCH

NP = S * TOP_K
NS = 16
CHUNK = NP // NS
HC = CHUNK // 2
SLOT_CH = N_SLOTS // NS


def _scatter_sc(pos_hbm, wf_hbm,
                rows_hbm, wv_hbm,
                pos_a, pos_b, wv_a, wv_b, rv_a, rv_b,
                zb_ref, zbf_ref, sh_rows, sh_wv):
    wid = lax.axis_index("s")
    lanes = lax.iota(jnp.int32, 16)

    pltpu.sync_copy(pos_hbm.at[pl.ds(wid * CHUNK, HC)], pos_a)
    pltpu.sync_copy(pos_hbm.at[pl.ds(wid * CHUNK + HC, HC)], pos_b)
    pltpu.sync_copy(wf_hbm.at[pl.ds(wid * CHUNK, HC)], wv_a)
    pltpu.sync_copy(wf_hbm.at[pl.ds(wid * CHUNK + HC, HC)], wv_b)

    for v in range(HC // 16):
        j0 = wid * CHUNK + v * 16
        rv_a[pl.ds(v * 16, 16)] = (j0 + lanes) >> 1
        rv_b[pl.ds(v * 16, 16)] = (j0 + HC + lanes) >> 1

    for v in range(SLOT_CH // 16):
        zb_ref[pl.ds(v * 16, 16)] = jnp.zeros((16,), jnp.int32)
        zbf_ref[pl.ds(v * 16, 16)] = jnp.zeros((16,), jnp.float32)
    pltpu.sync_copy(zb_ref, sh_rows.at[pl.ds(wid * SLOT_CH, SLOT_CH)])
    pltpu.sync_copy(zbf_ref, sh_wv.at[pl.ds(wid * SLOT_CH, SLOT_CH)])
    plsc.subcore_barrier()

    pltpu.sync_copy(rv_a, sh_rows.at[pos_a], add=True)
    pltpu.sync_copy(rv_b, sh_rows.at[pos_b], add=True)
    pltpu.sync_copy(wv_a, sh_wv.at[pos_a], add=True)
    pltpu.sync_copy(wv_b, sh_wv.at[pos_b], add=True)
    plsc.subcore_barrier()

    pltpu.sync_copy(sh_rows.at[pl.ds(wid * SLOT_CH, SLOT_CH)],
                    rows_hbm.at[pl.ds(wid * SLOT_CH, SLOT_CH)])
    pltpu.sync_copy(sh_wv.at[pl.ds(wid * SLOT_CH, SLOT_CH)],
                    wv_hbm.at[pl.ds(wid * SLOT_CH, SLOT_CH)])


_scatter = functools.partial(
    pl.kernel,
    out_type=(
        jax.ShapeDtypeStruct((N_SLOTS,), jnp.int32),
        jax.ShapeDtypeStruct((N_SLOTS,), jnp.float32),
    ),
    mesh=plsc.VectorSubcoreMesh(
        core_axis_name="c", subcore_axis_name="s", num_cores=1),
    scratch_types=(
        pltpu.VMEM((HC,), jnp.int32),
        pltpu.VMEM((HC,), jnp.int32),
        pltpu.VMEM((HC,), jnp.float32),
        pltpu.VMEM((HC,), jnp.float32),
        pltpu.VMEM((HC,), jnp.int32),
        pltpu.VMEM((HC,), jnp.int32),
        pltpu.VMEM((SLOT_CH,), jnp.int32),
        pltpu.VMEM((SLOT_CH,), jnp.float32),
        pltpu.VMEM_SHARED((N_SLOTS,), jnp.int32),
        pltpu.VMEM_SHARED((N_SLOTS,), jnp.float32),
    ),
)(_scatter_sc)


def _moe_block_kernel(eids_ref, rows_ref, nact_ref,
                      x_ref, w1_ref, b1_ref, w2_ref, b2_ref, wmat_ref,
                      out_ref, xg_ref, yacc_ref):
    b = pl.program_id(0)

    @pl.when(b == 0)
    def _init_out():
        out_ref[...] = jnp.zeros_like(out_ref)

    @pl.when(b < nact_ref[0])
    def _active():
        def gather(i, _):
            r = rows_ref[b * BM + i]
            xg_ref[pl.ds(i, 1), :] = x_ref[pl.ds(r, 1), :]
            return 0
        lax.fori_loop(0, BM, gather, 0, unroll=16)

        h = lax.dot_general(
            xg_ref[...].astype(jnp.bfloat16), w1_ref[0],
            (((1,), (1,)), ((), ())), preferred_element_type=jnp.float32)
        h = h + b1_ref[0, 0]
        h = 0.5 * h * (1.0 + lax.erf(h * 0.7071067811865476))
        y = lax.dot_general(
            h.astype(jnp.bfloat16), w2_ref[0],
            (((1,), (1,)), ((), ())), preferred_element_type=jnp.float32)
        yacc_ref[...] = (y + b2_ref[0]) * wmat_ref[0][:, 0:1]

        def scatter(i, _):
            r = rows_ref[b * BM + i]
            out_ref[pl.ds(r, 1), :] += yacc_ref[pl.ds(i, 1), :]
            return 0
        lax.fori_loop(0, BM, scatter, 0, unroll=16)


@functools.partial(jax.jit, static_argnums=())
def kernel(x, Wg, W1, b1, W2, b2):
    logits = jnp.einsum('bsh,eh->bse', jax.lax.stop_gradient(x), Wg)
    probs = jax.nn.softmax(logits, axis=-1)
    top_k_weights, top_k_indices = jax.lax.top_k(probs, TOP_K)

    e_flat = top_k_indices.reshape(NP).astype(jnp.int32)
    w_flat = top_k_weights.reshape(NP)
    onehot = (e_flat[:, None] == jnp.arange(E, dtype=jnp.int32)[None, :]
              ).astype(jnp.float32)
    ohc = onehot.reshape(NCH, RCH, E)
    ii = jax.lax.broadcasted_iota(jnp.int32, (RCH, RCH), 0)
    jj = jax.lax.broadcasted_iota(jnp.int32, (RCH, RCH), 1)
    ltri = (ii > jj).astype(jnp.float32)
    local_rank = jnp.einsum('ij,cje->cie', ltri, ohc,
                            preferred_element_type=jnp.float32)
    chunk_tot = ohc.sum(axis=1)
    chunk_pfx = jnp.concatenate(
        [jnp.zeros((1, E), jnp.float32), jnp.cumsum(chunk_tot[:-1], axis=0)])
    rank = (local_rank + chunk_pfx[:, None, :]).reshape(NP, E)
    counts = chunk_tot.sum(axis=0)
    bpe = jnp.ceil(counts / BM)
    block_start = jnp.concatenate(
        [jnp.zeros((1,), jnp.float32), jnp.cumsum(bpe)[:-1]])
    nactive = (block_start[E - 1] + bpe[E - 1]).astype(jnp.int32)
    slot_start = block_start * BM
    pos = ((slot_start[None, :] + rank) * onehot).sum(axis=1).astype(jnp.int32)
    eids = (jnp.arange(NB, dtype=jnp.float32)[:, None]
            >= block_start[None, 1:]).sum(axis=1).astype(jnp.int32)

    rows, wvals = _scatter(pos, w_flat)
    wmat = jnp.broadcast_to(
        wvals.reshape(NB, BM)[:, :, None], (NB, BM, 128))

    x2 = x.reshape(S, H)
    W1b = W1.astype(jnp.bfloat16)
    W2b = W2.astype(jnp.bfloat16)
    b1r = b1.reshape(E, 1, F)
    b2r = b2.reshape(E, 1, H)

    grid_spec = pltpu.PrefetchScalarGridSpec(
        num_scalar_prefetch=3,
        grid=(NB,),
        in_specs=[
            pl.BlockSpec((S, H), lambda b, eids, rows, nact: (0, 0)),
            pl.BlockSpec((1, F, H), lambda b, eids, rows, nact: (eids[b], 0, 0)),
            pl.BlockSpec((1, 1, F), lambda b, eids, rows, nact: (eids[b], 0, 0)),
            pl.BlockSpec((1, H, F), lambda b, eids, rows, nact: (eids[b], 0, 0)),
            pl.BlockSpec((1, 1, H), lambda b, eids, rows, nact: (eids[b], 0, 0)),
            pl.BlockSpec((1, BM, 128), lambda b, eids, rows, nact: (b, 0, 0)),
        ],
        out_specs=pl.BlockSpec((S, H), lambda b, eids, rows, nact: (0, 0)),
        scratch_shapes=[
            pltpu.VMEM((BM, H), jnp.float32),
            pltpu.VMEM((BM, H), jnp.float32),
        ],
    )

    out = pl.pallas_call(
        _moe_block_kernel,
        grid_spec=grid_spec,
        out_shape=jax.ShapeDtypeStruct((S, H), jnp.float32),
        compiler_params=pltpu.CompilerParams(
            dimension_semantics=("arbitrary",),
            vmem_limit_bytes=63 * 1024 * 1024,
        ),
    )(eids, rows, nactive.reshape(1), x2, W1b, b1r, W2b, b2r, wmat)

    return out.reshape(B, S, H)
